# baseline (speedup 1.0000x reference)
import functools

import jax
import jax.numpy as jnp
from jax import lax
from jax.experimental import pallas as pl
from jax.experimental.pallas import tpu as pltpu
from jax.experimental.pallas import tpu_sc as plsc

_N = 10000
_E = 320000
_B = 64
_K = 3
_TILES = 16
_CORES = 2
_CH = 128
_EPT = _E // _TILES
_NCH = 164
_SELF = 640
_TRASH = _NCH * _CH - _EPT - _SELF
_NR = 10240
_NACC = _NR
_RC = 32
_NZ = 20

_f32 = jnp.float32


def _mesh():
    return plsc.VectorSubcoreMesh(core_axis_name="c", subcore_axis_name="s")


def _deg_body(dstp, out, didx, obuf, zbuf, acc):
    c = lax.axis_index("c")
    s = lax.axis_index("s")
    pltpu.sync_copy(dstp.at[pl.ds(s * _NCH, _NCH), :], didx)

    def fill(i, _):
        obuf[i] = jnp.full((16,), 1.0, _f32)
        zbuf[i] = jnp.zeros((16,), _f32)
        return 0

    lax.fori_loop(0, _CH, fill, 0)
    for z in range(5):
        pltpu.sync_copy(zbuf, acc.at[pl.ds(s * 640 + z * 128, 128), :])
    plsc.subcore_barrier()
    lo = c * (_NCH // 2)
    hi = jnp.where(c == 0, _NCH // 2, _NCH)

    def chunk(j, _):
        pltpu.sync_copy(obuf, acc.at[didx.at[j]], add=True)
        return 0

    lax.fori_loop(lo, hi, chunk, 0)
    plsc.subcore_barrier()
    for z in range(5):
        r0 = s * 640 + z * 128
        pltpu.sync_copy(acc.at[pl.ds(r0, 128), :],
                        out.at[pl.ds(c * _NACC + r0, 128), :])


def _deg_call(dstp):
    f = pl.kernel(
        _deg_body,
        out_type=jax.ShapeDtypeStruct((_CORES * _NACC, 16), _f32),
        mesh=_mesh(),
        compiler_params=pltpu.CompilerParams(use_tc_tiling_on_sc=False),
        scratch_types=[
            pltpu.VMEM((_NCH, _CH), jnp.int32),
            pltpu.VMEM((_CH, 16), _f32),
            pltpu.VMEM((_CH, 16), _f32),
            pltpu.VMEM_SHARED((_NACC, 16), _f32),
        ],
    )
    return f(dstp)


def _prop_body(Ch, gin, srcp, dstp, d2rep, disrep, out, gscr,
               sidx, didx, upd, rbuf, drep, zbuf, acc, gsem, ssem,
               lsem, wsem):
    Chv = Ch // 16
    c = lax.axis_index("c")
    s = lax.axis_index("s")
    pltpu.sync_copy(srcp.at[pl.ds(s * _NCH, _NCH), :], sidx)
    pltpu.sync_copy(dstp.at[pl.ds(s * _NCH, _NCH), :], didx)
    coff = jnp.full((16,), c * _NR, dtype=jnp.int32)

    def offs(t, _):
        r = t // 8
        q = (t % 8) * 16
        sidx[r, pl.ds(q, 16)] = sidx[r, pl.ds(q, 16)] + coff
        return 0

    lax.fori_loop(0, _NCH * 8, offs, 0)

    def fz(t, _):
        zbuf[t // Chv, pl.ds((t % Chv) * 16, 16)] = jnp.zeros((16,), _f32)
        return 0

    lax.fori_loop(0, _RC * Chv, fz, 0)
    for z in range(_NZ):
        pltpu.sync_copy(zbuf, acc.at[pl.ds(s * 640 + z * _RC, _RC), :])
    plsc.subcore_barrier()

    for r in range(_K):
        gsrc = gin if r == 0 else gscr

        def iss_gather(j, sb):
            pltpu.async_copy(gsrc.at[sidx.at[j]], upd.at[sb], gsem.at[sb])

        def wait_gather(j, sb):
            pltpu.make_async_copy(gsrc.at[sidx.at[j]], upd.at[sb],
                                  gsem.at[sb]).wait()

        def iss_scat(j, sb):
            pltpu.async_copy(upd.at[sb], acc.at[didx.at[j]], ssem.at[sb],
                             add=True)

        def wait_scat(j, sb):
            pltpu.make_async_copy(upd.at[sb], acc.at[didx.at[j]],
                                  ssem.at[sb]).wait()

        iss_gather(0, 0)
        iss_gather(1, 1)

        def outer(it, _):
            base = it * 8
            for b in range(8):
                j = base + b
                sb = b % 4
                sg = (b + 2) % 4
                wait_gather(j, sb)
                iss_scat(j, sb)

                @pl.when(j >= 2)
                def _():
                    wait_scat(j - 2, sg)
                iss_gather(j + 2, sg)
            return 0

        lax.fori_loop(0, (_NCH - 4) // 8, outer, 0)
        for j in range(_NCH - 4, _NCH):
            sb = j % 4
            wait_gather(j, sb)
            iss_scat(j, sb)
            wait_scat(j - 2, (j + 2) % 4)
            if j + 2 < _NCH:
                iss_gather(j + 2, (j + 2) % 4)
        wait_scat(_NCH - 2, (_NCH - 2) % 4)
        wait_scat(_NCH - 1, (_NCH - 1) % 4)
        plsc.subcore_barrier()
        srep = d2rep if r < _K - 1 else disrep
        last = r == _K - 1
        gdst = out if last else gscr

        def iss_loads(k, slot):
            row0 = s * 640 + k * _RC
            pltpu.async_copy(acc.at[pl.ds(row0, _RC), :], rbuf.at[slot],
                             lsem.at[slot, 0])
            pltpu.async_copy(srep.at[pl.ds(row0, _RC), :], drep.at[slot],
                             lsem.at[slot, 1])

        def wait_loads(slot):
            row0 = s * 640
            pltpu.make_async_copy(acc.at[pl.ds(row0, _RC), :], rbuf.at[slot],
                                  lsem.at[slot, 0]).wait()
            pltpu.make_async_copy(srep.at[pl.ds(row0, _RC), :], drep.at[slot],
                                  lsem.at[slot, 1]).wait()

        def iss_writes(k, slot):
            row0 = s * 640 + k * _RC
            pltpu.async_copy(rbuf.at[slot],
                             gdst.at[pl.ds(c * _NR + row0, _RC), :],
                             wsem.at[slot, 0])
            if not last:
                pltpu.async_copy(zbuf, acc.at[pl.ds(row0, _RC), :],
                                 wsem.at[slot, 1])

        def wait_writes(slot):
            row0 = s * 640
            pltpu.make_async_copy(rbuf.at[slot],
                                  gdst.at[pl.ds(c * _NR + row0, _RC), :],
                                  wsem.at[slot, 0]).wait()
            if not last:
                pltpu.make_async_copy(zbuf, acc.at[pl.ds(row0, _RC), :],
                                      wsem.at[slot, 1]).wait()

        def compute(slot):
            def rrow(i, _):
                dv = drep[slot, i]
                for v in range(Chv):
                    rbuf[slot, i, pl.ds(v * 16, 16)] = (
                        dv * rbuf[slot, i, pl.ds(v * 16, 16)])
                return 0

            lax.fori_loop(0, _RC, rrow, 0)

        iss_loads(0, 0)

        def rpair(t, _):
            for half in (0, 1):
                k = 2 * t + half

                @pl.when(k >= 1)
                def _():
                    wait_writes(1 - half)

                @pl.when(k + 1 < _NZ)
                def _():
                    iss_loads(k + 1, 1 - half)
                wait_loads(half)
                compute(half)
                iss_writes(k, half)
            return 0

        lax.fori_loop(0, _NZ // 2, rpair, 0)
        wait_writes(1)
        if r < _K - 1:
            plsc.subcore_barrier()


def _prop_call(Ch, gin, srcp, dstp, d2rep, disrep):
    f = pl.kernel(
        functools.partial(_prop_body, Ch),
        out_type=[
            jax.ShapeDtypeStruct((_CORES * _NR, Ch), _f32),
            jax.ShapeDtypeStruct((_CORES * _NR, Ch), _f32),
        ],
        mesh=_mesh(),
        compiler_params=pltpu.CompilerParams(use_tc_tiling_on_sc=False),
        scratch_types=[
            pltpu.VMEM((_NCH, _CH), jnp.int32),
            pltpu.VMEM((_NCH, _CH), jnp.int32),
            pltpu.VMEM((4, _CH, Ch), _f32),
            pltpu.VMEM((2, _RC, Ch), _f32),
            pltpu.VMEM((2, _RC, 16), _f32),
            pltpu.VMEM((_RC, Ch), _f32),
            pltpu.VMEM_SHARED((_NACC, Ch), _f32),
            pltpu.SemaphoreType.DMA((4,)),
            pltpu.SemaphoreType.DMA((4,)),
            pltpu.SemaphoreType.DMA((2, 2)),
            pltpu.SemaphoreType.DMA((2, 2)),
        ],
    )
    return f(gin, srcp, dstp, d2rep, disrep)


_R = 400
_NSTEPS = _N // _R
_RP = 512
_PSTEPS = _NR // _RP


def _prep_body(pd0, pd1, x, g, d2r, disr):
    deg = pd0[:, 0:1] + pd1[:, 0:1]
    dis = lax.rsqrt(deg)
    d2 = 1.0 / deg
    gv = dis * x[...]
    g[...] = jnp.stack([gv[:, :64], gv[:, 64:]], axis=0)
    d2r[...] = jnp.broadcast_to(d2, (_RP, 16))
    disr[...] = jnp.broadcast_to(dis, (_RP, 16))


def _prep_call(pd0, pd1, x):
    return pl.pallas_call(
        _prep_body,
        grid=(_PSTEPS,),
        in_specs=[
            pl.BlockSpec((_RP, 16), lambda i: (i, 0)),
            pl.BlockSpec((_RP, 16), lambda i: (i, 0)),
            pl.BlockSpec((_RP, 128), lambda i: (i, 0)),
        ],
        out_specs=[
            pl.BlockSpec((2, _RP, 64), lambda i: (0, i, 0)),
            pl.BlockSpec((_RP, 16), lambda i: (i, 0)),
            pl.BlockSpec((_RP, 16), lambda i: (i, 0)),
        ],
        out_shape=[
            jax.ShapeDtypeStruct((2, _NR, 64), _f32),
            jax.ShapeDtypeStruct((_NR, 16), _f32),
            jax.ShapeDtypeStruct((_NR, 16), _f32),
        ],
    )(pd0, pd1, x)


def _mid_body(h1, bt, disr, W1, b1, W2, havg, gp, sums, cnt):
    i = pl.program_id(0)

    @pl.when(i == 0)
    def _():
        sums[...] = jnp.zeros_like(sums)
        cnt[...] = jnp.zeros_like(cnt)

    h1c = jnp.concatenate([h1[0], h1[1]], axis=1)
    h = jax.lax.dot_general(h1c, W1[...], (((1,), (1,)), ((), ())),
                            preferred_element_type=_f32) + b1[...]
    h = jnp.maximum(h, 0.0)
    b = bt[0, 0, :]
    valid = (i * _RP + lax.broadcasted_iota(jnp.int32, (_RP, 1), 0)) < _N
    oh = jnp.where(
        (b[:, None] == lax.broadcasted_iota(jnp.int32, (1, _B), 1)) & valid,
        1.0, 0.0)
    sums[...] += jax.lax.dot_general(oh, h, (((0,), (0,)), ((), ())),
                                     preferred_element_type=_f32)
    cnt[...] += jnp.broadcast_to(jnp.sum(oh, axis=0)[:, None], (_B, 256))
    p = jax.lax.dot_general(h, W2[...], (((1,), (1,)), ((), ())),
                            preferred_element_type=_f32)
    gpv = disr[:, 0:1] * p
    gp[...] = jnp.stack([gpv[:, :32], gpv[:, 32:]], axis=0)

    @pl.when(i == _PSTEPS - 1)
    def _():
        havg[...] = sums[...] / jnp.maximum(cnt[...], 1.0)


def _mid_call(h1, bt3, disr, W1, b1, W2):
    return pl.pallas_call(
        _mid_body,
        grid=(_PSTEPS,),
        in_specs=[
            pl.BlockSpec((2, _RP, 64), lambda i: (0, i, 0)),
            pl.BlockSpec((1, 1, _RP), lambda i: (i, 0, 0)),
            pl.BlockSpec((_RP, 16), lambda i: (i, 0)),
            pl.BlockSpec((256, 128), lambda i: (0, 0)),
            pl.BlockSpec((1, 256), lambda i: (0, 0)),
            pl.BlockSpec((64, 256), lambda i: (0, 0)),
        ],
        out_specs=[
            pl.BlockSpec((_B, 256), lambda i: (0, 0)),
            pl.BlockSpec((2, _RP, 32), lambda i: (0, i, 0)),
        ],
        out_shape=[
            jax.ShapeDtypeStruct((_B, 256), _f32),
            jax.ShapeDtypeStruct((2, _NR, 32), _f32),
        ],
        scratch_shapes=[
            pltpu.VMEM((_B, 256), _f32),
            pltpu.VMEM((_B, 256), _f32),
        ],
    )(h1, bt3, disr, W1, b1, W2)


def _fin_body(h2, b2, Wc1, bc1, Wc2, bc2, out):
    h2c = jnp.concatenate([h2[0], h2[1]], axis=1) + b2[...]
    h2c = jnp.maximum(h2c, 0.0)
    z = jax.lax.dot_general(h2c, Wc1[...], (((1,), (1,)), ((), ())),
                            preferred_element_type=_f32) + bc1[...]
    z = jnp.maximum(z, 0.0)
    out[...] = jax.lax.dot_general(z, Wc2[...], (((1,), (1,)), ((), ())),
                                   preferred_element_type=_f32) + bc2[...]


def _fin_call(h2, b2, Wc1, bc1, Wc2, bc2):
    return pl.pallas_call(
        _fin_body,
        grid=(_NSTEPS,),
        in_specs=[
            pl.BlockSpec((2, _R, 32), lambda i: (0, i, 0)),
            pl.BlockSpec((1, 64), lambda i: (0, 0)),
            pl.BlockSpec((16, 64), lambda i: (0, 0)),
            pl.BlockSpec((1, 16), lambda i: (0, 0)),
            pl.BlockSpec((16, 16), lambda i: (0, 0)),
            pl.BlockSpec((1, 16), lambda i: (0, 0)),
        ],
        out_specs=pl.BlockSpec((_R, 16), lambda i: (i, 0)),
        out_shape=jax.ShapeDtypeStruct((_N, 16), _f32),
    )(h2, b2, Wc1, bc1, Wc2, bc2)


def kernel(x, edge_index, batch, W1, b1, W2, b2, Wc1, bc1, Wc2, bc2):
    src = edge_index[0]
    dst = edge_index[1]
    srcr = src.reshape(_TILES, _EPT)
    dstr = dst.reshape(_TILES, _EPT)
    selfe = jnp.arange(_NR, dtype=jnp.int32).reshape(_TILES, _SELF)
    tidx = jnp.arange(_TRASH, dtype=jnp.int32)
    trash_src = jnp.broadcast_to(tidx % 16, (_TILES, _TRASH))
    trash_dst = jnp.broadcast_to(_N + (tidx % (_NACC - _N)), (_TILES, _TRASH))
    srcp = jnp.concatenate([srcr, selfe, trash_src],
                           axis=1).reshape(_TILES * _NCH, _CH)
    dstp = jnp.concatenate([dstr, selfe, trash_dst],
                           axis=1).reshape(_TILES * _NCH, _CH)

    pdeg = _deg_call(dstp)
    pd0 = pdeg[:_NACC]
    pd1 = pdeg[_NACC:]
    x_pad = jnp.pad(x, ((0, _NR - _N), (0, 0)))
    g0, d2rep, disrep = _prep_call(pd0, pd1, x_pad)
    h1, _ = _prop_call(64, g0.reshape(2 * _NR, 64), srcp, dstp, d2rep, disrep)
    bt3 = jnp.pad(batch, (0, _NR - _N)).reshape(_PSTEPS, 1, _RP)
    h_avg, gp = _mid_call(h1.reshape(2, _NR, 64), bt3, disrep, W1,
                          b1.reshape(1, 256), W2)
    h2, _ = _prop_call(32, gp.reshape(2 * _NR, 32), srcp, dstp,
                       d2rep, disrep)
    logits = _fin_call(h2.reshape(2, _NR, 32), b2.reshape(1, 64), Wc1,
                       bc1.reshape(1, 16), Wc2, bc2.reshape(1, 16))
    return (h_avg, logits)


# speedup vs baseline: 1.0949x; 1.0949x over previous
import functools

import jax
import jax.numpy as jnp
from jax import lax
from jax.experimental import pallas as pl
from jax.experimental.pallas import tpu as pltpu
from jax.experimental.pallas import tpu_sc as plsc

_N = 10000
_E = 320000
_B = 64
_K = 3
_TILES = 16
_CORES = 2
_CH = 128
_EPT = _E // _TILES
_NCH = 164
_SELF = 640
_TRASH = _NCH * _CH - _EPT - _SELF
_NR = 10240
_NACC = _NR
_RC = 32
_NZ = 20

_f32 = jnp.float32


def _mesh():
    return plsc.VectorSubcoreMesh(core_axis_name="c", subcore_axis_name="s")


def _deg_body(dstp, out, didx, obuf, zbuf, acc):
    c = lax.axis_index("c")
    s = lax.axis_index("s")
    pltpu.sync_copy(dstp.at[pl.ds(s * _NCH, _NCH), :], didx)

    def fill(i, _):
        obuf[i] = jnp.full((16,), 1.0, _f32)
        zbuf[i] = jnp.zeros((16,), _f32)
        return 0

    lax.fori_loop(0, _CH, fill, 0)
    for z in range(5):
        pltpu.sync_copy(zbuf, acc.at[pl.ds(s * 640 + z * 128, 128), :])
    plsc.subcore_barrier()
    lo = c * (_NCH // 2)
    hi = jnp.where(c == 0, _NCH // 2, _NCH)

    def chunk(j, _):
        pltpu.sync_copy(obuf, acc.at[didx.at[j]], add=True)
        return 0

    lax.fori_loop(lo, hi, chunk, 0)
    plsc.subcore_barrier()
    for z in range(5):
        r0 = s * 640 + z * 128
        pltpu.sync_copy(acc.at[pl.ds(r0, 128), :],
                        out.at[pl.ds(c * _NACC + r0, 128), :])


def _deg_call(dstp):
    f = pl.kernel(
        _deg_body,
        out_type=jax.ShapeDtypeStruct((_CORES * _NACC, 16), _f32),
        mesh=_mesh(),
        compiler_params=pltpu.CompilerParams(use_tc_tiling_on_sc=False),
        scratch_types=[
            pltpu.VMEM((_NCH, _CH), jnp.int32),
            pltpu.VMEM((_CH, 16), _f32),
            pltpu.VMEM((_CH, 16), _f32),
            pltpu.VMEM_SHARED((_NACC, 16), _f32),
        ],
    )
    return f(dstp)


def _prop_body(Ch, gin, srcp, dstp, d2rep, disrep, out, gscr,
               sidx, didx, upd, rbuf, drep, zbuf, acc, gsem, ssem,
               lsem, wsem):
    Chv = Ch // 16
    c = lax.axis_index("c")
    s = lax.axis_index("s")
    pltpu.sync_copy(srcp.at[pl.ds(s * _NCH, _NCH), :], sidx)
    pltpu.sync_copy(dstp.at[pl.ds(s * _NCH, _NCH), :], didx)
    coff = jnp.full((16,), c * _NR, dtype=jnp.int32)

    def offs(t, _):
        r = t // 8
        q = (t % 8) * 16
        sidx[r, pl.ds(q, 16)] = sidx[r, pl.ds(q, 16)] + coff
        return 0

    lax.fori_loop(0, _NCH * 8, offs, 0)

    def fz(t, _):
        zbuf[t // Chv, pl.ds((t % Chv) * 16, 16)] = jnp.zeros((16,), _f32)
        return 0

    lax.fori_loop(0, _RC * Chv, fz, 0)
    for z in range(_NZ):
        pltpu.sync_copy(zbuf, acc.at[pl.ds(s * 640 + z * _RC, _RC), :])
    plsc.subcore_barrier()

    nb = 4
    for r in range(_K):
        gsrc = gin if r == 0 else gscr
        for b in range(nb):
            pltpu.async_copy(gsrc.at[sidx.at[b]], upd.at[b], gsem.at[b])

        def outer(it, _):
            base = it * nb
            for b in range(nb):
                j = base + b
                pltpu.make_async_copy(gsrc.at[sidx.at[j]], upd.at[b],
                                      gsem.at[b]).wait()
                pltpu.async_copy(upd.at[b], acc.at[didx.at[j]], ssem.at[b],
                                 add=True)
            for b in range(nb):
                j2 = base + nb + b

                @pl.when(j2 < _NCH)
                def _():
                    pltpu.make_async_copy(upd.at[b], acc.at[didx.at[base + b]],
                                          ssem.at[b]).wait()
                    pltpu.async_copy(gsrc.at[sidx.at[j2]], upd.at[b],
                                     gsem.at[b])
            return 0

        lax.fori_loop(0, _NCH // nb, outer, 0)
        for b in range(nb):
            pltpu.make_async_copy(upd.at[b], acc.at[didx.at[_NCH - nb + b]],
                                  ssem.at[b]).wait()
        plsc.subcore_barrier()
        srep = d2rep if r < _K - 1 else disrep
        last = r == _K - 1
        gdst = out if last else gscr

        def iss_loads(k, slot):
            row0 = s * 640 + k * _RC
            pltpu.async_copy(acc.at[pl.ds(row0, _RC), :], rbuf.at[slot],
                             lsem.at[slot, 0])
            pltpu.async_copy(srep.at[pl.ds(row0, _RC), :], drep.at[slot],
                             lsem.at[slot, 1])

        def wait_loads(slot):
            row0 = s * 640
            pltpu.make_async_copy(acc.at[pl.ds(row0, _RC), :], rbuf.at[slot],
                                  lsem.at[slot, 0]).wait()
            pltpu.make_async_copy(srep.at[pl.ds(row0, _RC), :], drep.at[slot],
                                  lsem.at[slot, 1]).wait()

        def iss_writes(k, slot):
            row0 = s * 640 + k * _RC
            pltpu.async_copy(rbuf.at[slot],
                             gdst.at[pl.ds(c * _NR + row0, _RC), :],
                             wsem.at[slot, 0])
            if not last:
                pltpu.async_copy(zbuf, acc.at[pl.ds(row0, _RC), :],
                                 wsem.at[slot, 1])

        def wait_writes(slot):
            row0 = s * 640
            pltpu.make_async_copy(rbuf.at[slot],
                                  gdst.at[pl.ds(c * _NR + row0, _RC), :],
                                  wsem.at[slot, 0]).wait()
            if not last:
                pltpu.make_async_copy(zbuf, acc.at[pl.ds(row0, _RC), :],
                                      wsem.at[slot, 1]).wait()

        def compute(slot):
            def rrow(i, _):
                dv = drep[slot, i]
                for v in range(Chv):
                    rbuf[slot, i, pl.ds(v * 16, 16)] = (
                        dv * rbuf[slot, i, pl.ds(v * 16, 16)])
                return 0

            lax.fori_loop(0, _RC, rrow, 0)

        iss_loads(0, 0)

        def rpair(t, _):
            for half in (0, 1):
                k = 2 * t + half

                @pl.when(k >= 1)
                def _():
                    wait_writes(1 - half)

                @pl.when(k + 1 < _NZ)
                def _():
                    iss_loads(k + 1, 1 - half)
                wait_loads(half)
                compute(half)
                iss_writes(k, half)
            return 0

        lax.fori_loop(0, _NZ // 2, rpair, 0)
        wait_writes(1)
        if r < _K - 1:
            plsc.subcore_barrier()


def _prop_call(Ch, gin, srcp, dstp, d2rep, disrep):
    f = pl.kernel(
        functools.partial(_prop_body, Ch),
        out_type=[
            jax.ShapeDtypeStruct((_CORES * _NR, Ch), _f32),
            jax.ShapeDtypeStruct((_CORES * _NR, Ch), _f32),
        ],
        mesh=_mesh(),
        compiler_params=pltpu.CompilerParams(use_tc_tiling_on_sc=False),
        scratch_types=[
            pltpu.VMEM((_NCH, _CH), jnp.int32),
            pltpu.VMEM((_NCH, _CH), jnp.int32),
            pltpu.VMEM((4, _CH, Ch), _f32),
            pltpu.VMEM((2, _RC, Ch), _f32),
            pltpu.VMEM((2, _RC, 16), _f32),
            pltpu.VMEM((_RC, Ch), _f32),
            pltpu.VMEM_SHARED((_NACC, Ch), _f32),
            pltpu.SemaphoreType.DMA((4,)),
            pltpu.SemaphoreType.DMA((4,)),
            pltpu.SemaphoreType.DMA((2, 2)),
            pltpu.SemaphoreType.DMA((2, 2)),
        ],
    )
    return f(gin, srcp, dstp, d2rep, disrep)


_R = 400
_NSTEPS = _N // _R
_RP = 512
_PSTEPS = _NR // _RP


def _prep_body(pd0, pd1, x, g, d2r, disr):
    deg = pd0[:, 0:1] + pd1[:, 0:1]
    dis = lax.rsqrt(deg)
    d2 = 1.0 / deg
    gv = dis * x[...]
    g[...] = jnp.stack([gv[:, :64], gv[:, 64:]], axis=0)
    d2r[...] = jnp.broadcast_to(d2, (_RP, 16))
    disr[...] = jnp.broadcast_to(dis, (_RP, 16))


def _prep_call(pd0, pd1, x):
    return pl.pallas_call(
        _prep_body,
        grid=(_PSTEPS,),
        in_specs=[
            pl.BlockSpec((_RP, 16), lambda i: (i, 0)),
            pl.BlockSpec((_RP, 16), lambda i: (i, 0)),
            pl.BlockSpec((_RP, 128), lambda i: (i, 0)),
        ],
        out_specs=[
            pl.BlockSpec((2, _RP, 64), lambda i: (0, i, 0)),
            pl.BlockSpec((_RP, 16), lambda i: (i, 0)),
            pl.BlockSpec((_RP, 16), lambda i: (i, 0)),
        ],
        out_shape=[
            jax.ShapeDtypeStruct((2, _NR, 64), _f32),
            jax.ShapeDtypeStruct((_NR, 16), _f32),
            jax.ShapeDtypeStruct((_NR, 16), _f32),
        ],
    )(pd0, pd1, x)


def _mid_body(h1, bt, disr, W1, b1, W2, havg, gp, sums, cnt):
    i = pl.program_id(0)

    @pl.when(i == 0)
    def _():
        sums[...] = jnp.zeros_like(sums)
        cnt[...] = jnp.zeros_like(cnt)

    h1c = jnp.concatenate([h1[0], h1[1]], axis=1)
    h = jax.lax.dot_general(h1c, W1[...], (((1,), (1,)), ((), ())),
                            preferred_element_type=_f32) + b1[...]
    h = jnp.maximum(h, 0.0)
    b = bt[0, 0, :]
    valid = (i * _RP + lax.broadcasted_iota(jnp.int32, (_RP, 1), 0)) < _N
    oh = jnp.where(
        (b[:, None] == lax.broadcasted_iota(jnp.int32, (1, _B), 1)) & valid,
        1.0, 0.0)
    sums[...] += jax.lax.dot_general(oh, h, (((0,), (0,)), ((), ())),
                                     preferred_element_type=_f32)
    cnt[...] += jnp.broadcast_to(jnp.sum(oh, axis=0)[:, None], (_B, 256))
    p = jax.lax.dot_general(h, W2[...], (((1,), (1,)), ((), ())),
                            preferred_element_type=_f32)
    gpv = disr[:, 0:1] * p
    gp[...] = jnp.stack([gpv[:, :32], gpv[:, 32:]], axis=0)

    @pl.when(i == _PSTEPS - 1)
    def _():
        havg[...] = sums[...] / jnp.maximum(cnt[...], 1.0)


def _mid_call(h1, bt3, disr, W1, b1, W2):
    return pl.pallas_call(
        _mid_body,
        grid=(_PSTEPS,),
        in_specs=[
            pl.BlockSpec((2, _RP, 64), lambda i: (0, i, 0)),
            pl.BlockSpec((1, 1, _RP), lambda i: (i, 0, 0)),
            pl.BlockSpec((_RP, 16), lambda i: (i, 0)),
            pl.BlockSpec((256, 128), lambda i: (0, 0)),
            pl.BlockSpec((1, 256), lambda i: (0, 0)),
            pl.BlockSpec((64, 256), lambda i: (0, 0)),
        ],
        out_specs=[
            pl.BlockSpec((_B, 256), lambda i: (0, 0)),
            pl.BlockSpec((2, _RP, 32), lambda i: (0, i, 0)),
        ],
        out_shape=[
            jax.ShapeDtypeStruct((_B, 256), _f32),
            jax.ShapeDtypeStruct((2, _NR, 32), _f32),
        ],
        scratch_shapes=[
            pltpu.VMEM((_B, 256), _f32),
            pltpu.VMEM((_B, 256), _f32),
        ],
    )(h1, bt3, disr, W1, b1, W2)


def _fin_body(h2, b2, Wc1, bc1, Wc2, bc2, out):
    h2c = jnp.concatenate([h2[0], h2[1]], axis=1) + b2[...]
    h2c = jnp.maximum(h2c, 0.0)
    z = jax.lax.dot_general(h2c, Wc1[...], (((1,), (1,)), ((), ())),
                            preferred_element_type=_f32) + bc1[...]
    z = jnp.maximum(z, 0.0)
    out[...] = jax.lax.dot_general(z, Wc2[...], (((1,), (1,)), ((), ())),
                                   preferred_element_type=_f32) + bc2[...]


def _fin_call(h2, b2, Wc1, bc1, Wc2, bc2):
    return pl.pallas_call(
        _fin_body,
        grid=(_NSTEPS,),
        in_specs=[
            pl.BlockSpec((2, _R, 32), lambda i: (0, i, 0)),
            pl.BlockSpec((1, 64), lambda i: (0, 0)),
            pl.BlockSpec((16, 64), lambda i: (0, 0)),
            pl.BlockSpec((1, 16), lambda i: (0, 0)),
            pl.BlockSpec((16, 16), lambda i: (0, 0)),
            pl.BlockSpec((1, 16), lambda i: (0, 0)),
        ],
        out_specs=pl.BlockSpec((_R, 16), lambda i: (i, 0)),
        out_shape=jax.ShapeDtypeStruct((_N, 16), _f32),
    )(h2, b2, Wc1, bc1, Wc2, bc2)


def kernel(x, edge_index, batch, W1, b1, W2, b2, Wc1, bc1, Wc2, bc2):
    src = edge_index[0]
    dst = edge_index[1]
    srcr = src.reshape(_TILES, _EPT)
    dstr = dst.reshape(_TILES, _EPT)
    selfe = jnp.arange(_NR, dtype=jnp.int32).reshape(_TILES, _SELF)
    tidx = jnp.arange(_TRASH, dtype=jnp.int32)
    trash_src = jnp.broadcast_to(tidx % 16, (_TILES, _TRASH))
    trash_dst = jnp.broadcast_to(_N + (tidx % (_NACC - _N)), (_TILES, _TRASH))
    srcp = jnp.concatenate([srcr, selfe, trash_src],
                           axis=1).reshape(_TILES * _NCH, _CH)
    dstp = jnp.concatenate([dstr, selfe, trash_dst],
                           axis=1).reshape(_TILES * _NCH, _CH)

    pdeg = _deg_call(dstp)
    pd0 = pdeg[:_NACC]
    pd1 = pdeg[_NACC:]
    x_pad = jnp.pad(x, ((0, _NR - _N), (0, 0)))
    g0, d2rep, disrep = _prep_call(pd0, pd1, x_pad)
    h1, _ = _prop_call(64, g0.reshape(2 * _NR, 64), srcp, dstp, d2rep, disrep)
    bt3 = jnp.pad(batch, (0, _NR - _N)).reshape(_PSTEPS, 1, _RP)
    h_avg, gp = _mid_call(h1.reshape(2, _NR, 64), bt3, disrep, W1,
                          b1.reshape(1, 256), W2)
    h2, _ = _prop_call(32, gp.reshape(2 * _NR, 32), srcp, dstp,
                       d2rep, disrep)
    logits = _fin_call(h2.reshape(2, _NR, 32), b2.reshape(1, 64), Wc1,
                       bc1.reshape(1, 16), Wc2, bc2.reshape(1, 16))
    return (h_avg, logits)
